# baseline (device time: 12098 ns/iter reference)
import jax
import jax.numpy as jnp
from jax import lax
from jax.experimental import pallas as pl
from jax.experimental.pallas import tpu as pltpu

N_DEV = 8
N_CHUNK = 4
EPS = 1e-5


def kernel(x, gamma, beta):
    m, n_local = x.shape
    n_global = n_local * N_DEV
    mc = m // N_CHUNK
    rc = mc // 128

    def body(x_ref, gamma_ref, beta_ref, out_ref,
             xv, ov, comm_ref, load_sems, store_sems, send_sems, recv_sems):
        my = lax.axis_index("i")

        barrier_sem = pltpu.get_barrier_semaphore()
        for d in range(1, N_DEV):
            peer = lax.rem(my + d, N_DEV)
            pl.semaphore_signal(
                barrier_sem, inc=1,
                device_id=(peer,), device_id_type=pl.DeviceIdType.MESH,
            )

        loads = []
        for c in range(N_CHUNK):
            cp = pltpu.make_async_copy(
                x_ref.at[pl.ds(c * mc, mc), :],
                xv.at[pl.ds(c * mc, mc), :],
                load_sems.at[c],
            )
            cp.start()
            loads.append(cp)

        for c in range(N_CHUNK):
            loads[c].wait()
            xc = xv[c * mc:(c + 1) * mc, :]
            comm_ref[0, c * rc:(c + 1) * rc, :] = (
                jnp.sum(xc, axis=1).reshape(rc, 128))
            comm_ref[0, 8 + c * rc:8 + (c + 1) * rc, :] = (
                jnp.sum(xc * xc, axis=1).reshape(rc, 128))

        pl.semaphore_wait(barrier_sem, N_DEV - 1)

        rdmas = []
        for d in range(1, N_DEV):
            peer = lax.rem(my + d, N_DEV)
            r = pltpu.make_async_remote_copy(
                src_ref=comm_ref.at[0],
                dst_ref=comm_ref.at[d],
                send_sem=send_sems.at[d],
                recv_sem=recv_sems.at[d],
                device_id=(peer,),
                device_id_type=pl.DeviceIdType.MESH,
            )
            r.start()
            rdmas.append(r)
        for r in rdmas:
            r.wait()

        tot = comm_ref[0, :, :]
        for d in range(1, N_DEV):
            tot = tot + comm_ref[d, :, :]
        stores = []
        for c in range(N_CHUNK):
            mean = tot[c * rc:(c + 1) * rc, :].reshape(mc) / n_global
            msq = tot[8 + c * rc:8 + (c + 1) * rc, :].reshape(mc) / n_global
            inv = lax.rsqrt(msq - mean * mean + EPS)
            xc = xv[c * mc:(c + 1) * mc, :]
            ov[c * mc:(c + 1) * mc, :] = (
                gamma_ref[:, :] * ((xc - mean[:, None]) * inv[:, None])
                + beta_ref[:, :]
            )
            st = pltpu.make_async_copy(
                ov.at[pl.ds(c * mc, mc), :],
                out_ref.at[pl.ds(c * mc, mc), :],
                store_sems.at[c],
            )
            st.start()
            stores.append(st)
        for st in stores:
            st.wait()

    g2 = gamma.reshape(1, n_local)
    b2 = beta.reshape(1, n_local)
    return pl.pallas_call(
        body,
        out_shape=jax.ShapeDtypeStruct((m, n_local), x.dtype),
        in_specs=[
            pl.BlockSpec(memory_space=pl.ANY),
            pl.BlockSpec(memory_space=pltpu.VMEM),
            pl.BlockSpec(memory_space=pltpu.VMEM),
        ],
        out_specs=pl.BlockSpec(memory_space=pl.ANY),
        scratch_shapes=[
            pltpu.VMEM((m, n_local), jnp.float32),
            pltpu.VMEM((m, n_local), jnp.float32),
            pltpu.VMEM((N_DEV, 16, 128), jnp.float32),
            pltpu.SemaphoreType.DMA((N_CHUNK,)),
            pltpu.SemaphoreType.DMA((N_CHUNK,)),
            pltpu.SemaphoreType.DMA((N_DEV,)),
            pltpu.SemaphoreType.DMA((N_DEV,)),
        ],
        compiler_params=pltpu.CompilerParams(collective_id=0),
    )(x, g2, b2)


# device time: 11339 ns/iter; 1.0669x vs baseline; 1.0669x over previous
import jax
import jax.numpy as jnp
from jax import lax
from jax.experimental import pallas as pl
from jax.experimental.pallas import tpu as pltpu

N_DEV = 8
EPS = 1e-5


def kernel(x, gamma, beta):
    m, n_local = x.shape
    n_global = n_local * N_DEV

    def body(x_ref, gamma_ref, beta_ref, out_ref, comm_ref, send_sems, recv_sems):
        my = lax.axis_index("i")

        barrier_sem = pltpu.get_barrier_semaphore()
        for d in range(1, N_DEV):
            peer = lax.rem(my + d, N_DEV)
            pl.semaphore_signal(
                barrier_sem, inc=1,
                device_id=(peer,), device_id_type=pl.DeviceIdType.MESH,
            )
        pl.semaphore_wait(barrier_sem, N_DEV - 1)

        xs = x_ref[:, :]
        comm_ref[0, 0:8, :] = jnp.sum(xs, axis=1).reshape(8, 128)
        comm_ref[0, 8:16, :] = jnp.sum(xs * xs, axis=1).reshape(8, 128)

        rdmas = []
        for d in range(1, N_DEV):
            peer = lax.rem(my + d, N_DEV)
            rdma = pltpu.make_async_remote_copy(
                src_ref=comm_ref.at[0],
                dst_ref=comm_ref.at[d],
                send_sem=send_sems.at[d],
                recv_sem=recv_sems.at[d],
                device_id=(peer,),
                device_id_type=pl.DeviceIdType.MESH,
            )
            rdma.start()
            rdmas.append(rdma)
        for rdma in rdmas:
            rdma.wait()

        tot = comm_ref[0, :, :]
        for d in range(1, N_DEV):
            tot = tot + comm_ref[d, :, :]
        mean = tot[0:8, :].reshape(m) / n_global
        var = tot[8:16, :].reshape(m) / n_global - mean * mean
        inv = lax.rsqrt(var + EPS)
        scale = inv[:, None]
        shift = mean[:, None]
        out_ref[:, :] = gamma_ref[:, :] * ((xs - shift) * scale) + beta_ref[:, :]

    g2 = gamma.reshape(1, n_local)
    b2 = beta.reshape(1, n_local)
    return pl.pallas_call(
        body,
        out_shape=jax.ShapeDtypeStruct((m, n_local), x.dtype),
        in_specs=[pl.BlockSpec(memory_space=pltpu.VMEM)] * 3,
        out_specs=pl.BlockSpec(memory_space=pltpu.VMEM),
        scratch_shapes=[
            pltpu.VMEM((N_DEV, 16, 128), jnp.float32),
            pltpu.SemaphoreType.DMA((N_DEV,)),
            pltpu.SemaphoreType.DMA((N_DEV,)),
        ],
        compiler_params=pltpu.CompilerParams(collective_id=0),
    )(x, g2, b2)
